# Initial kernel scaffold; baseline (speedup 1.0000x reference)
import jax
import jax.numpy as jnp
from jax.experimental import pallas as pl


def kernel(x, W1, b1, gamma, beta, W2, b2):
    raise NotImplementedError("write your pallas kernel here")


# speedup vs baseline: 3.3445x; 3.3445x over previous
import functools

import jax
import jax.numpy as jnp
from jax.experimental import pallas as pl
from jax.experimental.pallas import tpu as pltpu

TOPK = 8
BN_EPS = 1e-5


def _mlp_stats_body(x_ref, w1_ref, b1_ref, h_ref, stats_ref):
    i = pl.program_id(0)
    h = jnp.dot(x_ref[...], w1_ref[...], preferred_element_type=jnp.float32)
    h = jnp.maximum(h + b1_ref[...][None, :], 0.0)
    h_ref[...] = h

    s = jnp.sum(h, axis=0)
    ss = jnp.sum(h * h, axis=0)
    upd = jnp.concatenate(
        [s[None, :], ss[None, :], jnp.zeros((6, s.shape[0]), jnp.float32)], axis=0
    )

    @pl.when(i == 0)
    def _():
        stats_ref[...] = jnp.zeros_like(stats_ref)

    stats_ref[...] += upd


def _router_body(nrows, h_ref, stats_ref, gamma_ref, beta_ref, w2_ref, b2_ref,
                 out_ref):
    inv_n = 1.0 / nrows
    mean = stats_ref[0, :] * inv_n
    var = stats_ref[1, :] * inv_n - mean * mean
    rstd = 1.0 / jnp.sqrt(var + BN_EPS)
    scale = gamma_ref[...] * rstd
    shift = beta_ref[...] - mean * scale

    hn = h_ref[...] * scale[None, :] + shift[None, :]
    logits = jnp.dot(hn, w2_ref[...], preferred_element_type=jnp.float32)
    logits = logits + b2_ref[...]

    e = logits.shape[1]
    iota = jax.lax.broadcasted_iota(jnp.int32, logits.shape, 1)
    work = logits
    mask = jnp.zeros(logits.shape, jnp.bool_)
    m0 = None
    for k in range(TOPK):
        m = jnp.max(work, axis=1, keepdims=True)
        if k == 0:
            m0 = m
        is_m = work == m
        am = jnp.min(jnp.where(is_m, iota, e), axis=1, keepdims=True)
        sel = iota == am
        mask = mask | sel
        work = jnp.where(sel, -jnp.inf, work)

    p = jnp.where(mask, jnp.exp(logits - m0), 0.0)
    out_ref[...] = p / jnp.sum(p, axis=1, keepdims=True)


def kernel(x, W1, b1, gamma, beta, W2, b2):
    B, D = x.shape
    H = W1.shape[1]
    E = W2.shape[1]

    TB_A = 256
    grid_a = B // TB_A
    h, stats = pl.pallas_call(
        _mlp_stats_body,
        grid=(grid_a,),
        in_specs=[
            pl.BlockSpec((TB_A, D), lambda i: (i, 0)),
            pl.BlockSpec((D, H), lambda i: (0, 0)),
            pl.BlockSpec((H,), lambda i: (0,)),
        ],
        out_specs=[
            pl.BlockSpec((TB_A, H), lambda i: (i, 0)),
            pl.BlockSpec((8, H), lambda i: (0, 0)),
        ],
        out_shape=[
            jax.ShapeDtypeStruct((B, H), jnp.float32),
            jax.ShapeDtypeStruct((8, H), jnp.float32),
        ],
        compiler_params=pltpu.CompilerParams(
            dimension_semantics=("arbitrary",),
        ),
    )(x, W1, b1)

    TB_B = 512
    grid_b = B // TB_B
    out = pl.pallas_call(
        functools.partial(_router_body, float(B)),
        grid=(grid_b,),
        in_specs=[
            pl.BlockSpec((TB_B, H), lambda i: (i, 0)),
            pl.BlockSpec((8, H), lambda i: (0, 0)),
            pl.BlockSpec((H,), lambda i: (0,)),
            pl.BlockSpec((H,), lambda i: (0,)),
            pl.BlockSpec((H, E), lambda i: (0, 0)),
            pl.BlockSpec((1, E), lambda i: (0, 0)),
        ],
        out_specs=pl.BlockSpec((TB_B, E), lambda i: (i, 0)),
        out_shape=jax.ShapeDtypeStruct((B, E), jnp.float32),
        compiler_params=pltpu.CompilerParams(
            dimension_semantics=("arbitrary",),
        ),
    )(h, stats, gamma, beta, W2, b2[None, :])
    return out
